# Initial kernel scaffold; baseline (speedup 1.0000x reference)
import jax
import jax.numpy as jnp
from jax.experimental import pallas as pl


def kernel(x, edge_index, batch, W1, b1, W2, b2, Wp, bp, qw, Wc1, bc1, Wc2, bc2):
    raise NotImplementedError("write your pallas kernel here")


# speedup vs baseline: 13.9733x; 13.9733x over previous
import functools

import jax
import jax.numpy as jnp
import numpy as np
from jax import lax
from jax.experimental import pallas as pl
from jax.experimental.pallas import tpu as pltpu
from jax.experimental.pallas import tpu_sc as plsc

N = 10000
NPAD = 10240
E = 320000
D = 128
H = 64
Z = 32
B = 128
NQ = 4
NL = 2

NC = 2
NS = 16
NW = NC * NS
CHUNK = 128
CPT = 80
EPT = CPT * CHUNK
EPAD = NW * EPT
DUMMY = N
RPT = NPAD // NS
RB = 256
GRID = NPAD // RB

_f32 = jnp.float32


def _perm_mat(perm):
    P = np.zeros((16, 16), np.float32)
    for k in range(16):
        P[k, perm(k)] = 1.0
    return P


def _bit(w):
    return 1 << (NQ - 1 - w)


_PW = np.stack([_perm_mat(lambda k, w=w: k ^ _bit(w)) for w in range(NQ)])
_SGN = np.array([[1.0 if (j & _bit(w)) else -1.0 for j in range(16)]
                 for w in range(NQ)], np.float32)
_CN = np.stack([_perm_mat(lambda k, c=c: (k ^ _bit(c + 1)) if (k & _bit(c)) else k)
                for c in range(NQ - 1)])
_MEAS = np.array([[1.0 - 2.0 * ((j >> (NQ - 1 - i)) & 1) for i in range(NQ)]
                  for j in range(16)], np.float32)
_ONEHOT0 = np.zeros((CHUNK, 16), np.float32)
_ONEHOT0[:, 0] = 1.0


def _sc_degree(dstm, ones16, zeros16):
    mesh = plsc.VectorSubcoreMesh(core_axis_name="c", subcore_axis_name="s")

    @functools.partial(
        pl.kernel, mesh=mesh,
        compiler_params=pltpu.CompilerParams(use_tc_tiling_on_sc=False),
        out_type=jax.ShapeDtypeStruct((NC, NPAD, 16), _f32),
        scratch_types=[
            pltpu.VMEM((CPT, CHUNK), jnp.int32),
            pltpu.VMEM((CHUNK, 16), _f32),
            pltpu.VMEM((CHUNK, 16), _f32),
            pltpu.VMEM_SHARED((NPAD, 16), _f32),
        ])
    def deg_kernel(dst_hbm, ones_hbm, zeros_hbm, out_hbm, dst_v, ones_v, z_v, acc_sh):
        c = lax.axis_index("c")
        s = lax.axis_index("s")
        wid = s * NC + c
        pltpu.sync_copy(dst_hbm.at[pl.ds(wid * CPT, CPT)], dst_v)
        pltpu.sync_copy(ones_hbm, ones_v)
        pltpu.sync_copy(zeros_hbm, z_v)
        for r in range(RPT // CHUNK):
            pltpu.sync_copy(z_v, acc_sh.at[pl.ds((s * (RPT // CHUNK) + r) * CHUNK, CHUNK)])
        plsc.subcore_barrier()

        def body(j, carry):
            pltpu.sync_copy(ones_v, acc_sh.at[dst_v.at[j]], add=True)
            return carry

        lax.fori_loop(0, CPT, body, 0)
        plsc.subcore_barrier()
        pltpu.sync_copy(acc_sh.at[pl.ds(s * RPT, RPT)],
                        out_hbm.at[c, pl.ds(s * RPT, RPT)])

    return deg_kernel(dstm, ones16, zeros16)


def _sc_aggregate(srcm, dstm, table, zeros, width):
    mesh = plsc.VectorSubcoreMesh(core_axis_name="c", subcore_axis_name="s")

    @functools.partial(
        pl.kernel, mesh=mesh,
        compiler_params=pltpu.CompilerParams(use_tc_tiling_on_sc=False),
        out_type=jax.ShapeDtypeStruct((NC, NPAD, width), _f32),
        scratch_types=[
            pltpu.VMEM((CPT, CHUNK), jnp.int32),
            pltpu.VMEM((CPT, CHUNK), jnp.int32),
            pltpu.VMEM((CHUNK, width), _f32),
            pltpu.VMEM_SHARED((NPAD, width), _f32),
            pltpu.SemaphoreType.DMA,
        ])
    def agg_kernel(src_hbm, dst_hbm, table_hbm, zeros_hbm, out_hbm,
                   src_v, dst_v, buf_v, acc_sh, sem):
        c = lax.axis_index("c")
        s = lax.axis_index("s")
        wid = s * NC + c
        pltpu.sync_copy(src_hbm.at[pl.ds(wid * CPT, CPT)], src_v)
        pltpu.sync_copy(dst_hbm.at[pl.ds(wid * CPT, CPT)], dst_v)
        pltpu.sync_copy(zeros_hbm, buf_v)
        for r in range(RPT // CHUNK):
            pltpu.sync_copy(buf_v, acc_sh.at[pl.ds((s * (RPT // CHUNK) + r) * CHUNK, CHUNK)])
        plsc.subcore_barrier()

        def body(j, carry):
            pltpu.async_copy(table_hbm.at[src_v.at[j]], buf_v, sem).wait()
            pltpu.sync_copy(buf_v, acc_sh.at[dst_v.at[j]], add=True)
            return carry

        lax.fori_loop(0, CPT, body, 0)
        plsc.subcore_barrier()
        pltpu.sync_copy(acc_sh.at[pl.ds(s * RPT, RPT)],
                        out_hbm.at[c, pl.ds(s * RPT, RPT)])

    return agg_kernel(srcm, dstm, table, zeros)


def _tc_encode(x_pad, W1, degw):
    def body(x_ref, w_ref, deg_ref, hw_ref, dinv_ref):
        deg = deg_ref[0] + deg_ref[1]
        deg = jnp.sum(deg, axis=1) + 1.0
        dinv = lax.rsqrt(jnp.maximum(deg, 1.0))[:, None]
        hw = lax.dot_general(x_ref[...], w_ref[...], (((1,), (0,)), ((), ())),
                             preferred_element_type=_f32,
                             precision=lax.Precision.HIGHEST)
        hw_ref[...] = hw * dinv
        dinv_ref[...] = jnp.broadcast_to(dinv, (RB, 16))

    return pl.pallas_call(
        body,
        grid=(GRID,),
        in_specs=[pl.BlockSpec((RB, D), lambda i: (i, 0)),
                  pl.BlockSpec((D, H), lambda i: (0, 0)),
                  pl.BlockSpec((NC, RB, 16), lambda i: (0, i, 0))],
        out_specs=[pl.BlockSpec((RB, H), lambda i: (i, 0)),
                   pl.BlockSpec((RB, 16), lambda i: (i, 0))],
        out_shape=[jax.ShapeDtypeStruct((NPAD, H), _f32),
                   jax.ShapeDtypeStruct((NPAD, 16), _f32)],
    )(x_pad, W1, degw)


def _tc_mid(p1, hw1p, dinv16, b1r, W2):
    def body(p_ref, hw_ref, dinv_ref, b_ref, w_ref, out_ref):
        dinv = dinv_ref[...][:, :1]
        agg = p_ref[0] + p_ref[1] + hw_ref[...]
        h1 = jnp.maximum(agg * dinv + b_ref[...], 0.0)
        out_ref[...] = lax.dot_general(h1, w_ref[...], (((1,), (0,)), ((), ())),
                                       preferred_element_type=_f32,
                             precision=lax.Precision.HIGHEST) * dinv

    return pl.pallas_call(
        body,
        grid=(GRID,),
        in_specs=[pl.BlockSpec((NC, RB, H), lambda i: (0, i, 0)),
                  pl.BlockSpec((RB, H), lambda i: (i, 0)),
                  pl.BlockSpec((RB, 16), lambda i: (i, 0)),
                  pl.BlockSpec((1, H), lambda i: (0, 0)),
                  pl.BlockSpec((H, Z), lambda i: (0, 0))],
        out_specs=pl.BlockSpec((RB, Z), lambda i: (i, 0)),
        out_shape=jax.ShapeDtypeStruct((NPAD, Z), _f32),
    )(p1, hw1p, dinv16, b1r, W2)


def _tc_pool(p2, hw2p, dinv16, b2r, batch3):
    def body(p_ref, hw_ref, dinv_ref, b_ref, bt_ref, sums_ref, cnts_ref):
        i = pl.program_id(0)
        dinv = dinv_ref[...][:, :1]
        h2 = (p_ref[0] + p_ref[1] + hw_ref[...]) * dinv + b_ref[...]
        bcol = bt_ref[0]
        iot = lax.broadcasted_iota(jnp.int32, (RB, B), 1)
        oh = (bcol == iot).astype(_f32)
        ps = lax.dot_general(oh, h2, (((0,), (0,)), ((), ())),
                             preferred_element_type=_f32,
                             precision=lax.Precision.HIGHEST)
        pc = lax.dot_general(oh, jnp.ones((RB, Z), _f32), (((0,), (0,)), ((), ())),
                             preferred_element_type=_f32,
                             precision=lax.Precision.HIGHEST)

        @pl.when(i == 0)
        def _():
            sums_ref[...] = jnp.zeros_like(sums_ref)
            cnts_ref[...] = jnp.zeros_like(cnts_ref)

        sums_ref[...] += ps
        cnts_ref[...] += pc

    return pl.pallas_call(
        body,
        grid=(GRID,),
        in_specs=[pl.BlockSpec((NC, RB, Z), lambda i: (0, i, 0)),
                  pl.BlockSpec((RB, Z), lambda i: (i, 0)),
                  pl.BlockSpec((RB, 16), lambda i: (i, 0)),
                  pl.BlockSpec((1, Z), lambda i: (0, 0)),
                  pl.BlockSpec((1, RB, 1), lambda i: (i, 0, 0))],
        out_specs=[pl.BlockSpec((B, Z), lambda i: (0, 0)),
                   pl.BlockSpec((B, Z), lambda i: (0, 0))],
        out_shape=[jax.ShapeDtypeStruct((B, Z), _f32),
                   jax.ShapeDtypeStruct((B, Z), _f32)],
    )(p2, hw2p, dinv16, b2r, batch3)


def _tc_head(sums, cnts, Wp, bpr, qw, pw, sgn, cn, meas, Wc1a, Wc1b, bc1r, Wc2, bc2r):
    def body(sums_ref, cnts_ref, wp_ref, bp_ref, qw_ref, pw_ref, sgn_ref,
             cn_ref, m_ref, w1a_ref, w1b_ref, b1_ref, w2_ref, b2_ref, out_ref):
        emb = sums_ref[...] / jnp.maximum(cnts_ref[...], 1.0)
        q_in = jnp.tanh(lax.dot_general(emb, wp_ref[...], (((1,), (0,)), ((), ())),
                                        preferred_element_type=_f32,
                             precision=lax.Precision.HIGHEST)
                        + bp_ref[...]) * 3.14159
        pw_all = pw_ref[...]
        sgn_all = sgn_ref[...]
        cn_all = cn_ref[...]
        qw_all = qw_ref[...]
        col = lax.broadcasted_iota(jnp.int32, (B, 16), 1)
        state = jnp.where(col == 0, 1.0, 0.0).astype(_f32)

        def ry(state, c, s, w):
            rot = lax.dot_general(state, pw_all[w], (((1,), (0,)), ((), ())),
                                  preferred_element_type=_f32,
                             precision=lax.Precision.HIGHEST)
            return c * state + (s * sgn_all[w:w + 1, :]) * rot

        for w in range(NQ):
            th = q_in[:, w:w + 1] * 0.5
            state = ry(state, jnp.cos(th), jnp.sin(th), w)
        for l in range(NL):
            for w in range(NQ):
                th = qw_all[l:l + 1, w:w + 1] * 0.5
                state = ry(state, jnp.cos(th), jnp.sin(th), w)
            for k in range(NQ - 1):
                state = lax.dot_general(state, cn_all[k], (((1,), (0,)), ((), ())),
                                        preferred_element_type=_f32,
                             precision=lax.Precision.HIGHEST)
        q_out = lax.dot_general(state * state, m_ref[...], (((1,), (0,)), ((), ())),
                                preferred_element_type=_f32,
                             precision=lax.Precision.HIGHEST)
        hid = jnp.maximum(
            lax.dot_general(emb, w1a_ref[...], (((1,), (0,)), ((), ())),
                            preferred_element_type=_f32,
                             precision=lax.Precision.HIGHEST)
            + lax.dot_general(q_out, w1b_ref[...], (((1,), (0,)), ((), ())),
                              preferred_element_type=_f32,
                             precision=lax.Precision.HIGHEST)
            + b1_ref[...], 0.0)
        out_ref[...] = lax.dot_general(hid, w2_ref[...], (((1,), (0,)), ((), ())),
                                       preferred_element_type=_f32,
                             precision=lax.Precision.HIGHEST) + b2_ref[...]

    return pl.pallas_call(
        body,
        out_shape=jax.ShapeDtypeStruct((B, 1), _f32),
    )(sums, cnts, Wp, bpr, qw, pw, sgn, cn, meas, Wc1a, Wc1b, bc1r, Wc2, bc2r)


def kernel(x, edge_index, batch, W1, b1, W2, b2, Wp, bp, qw, Wc1, bc1, Wc2, bc2):
    i32 = jnp.int32
    pad = EPAD - E
    srcm = jnp.concatenate([edge_index[0].astype(i32),
                            jnp.full((pad,), DUMMY, i32)]).reshape(EPAD // CHUNK, CHUNK)
    dstm = jnp.concatenate([edge_index[1].astype(i32),
                            jnp.full((pad,), DUMMY, i32)]).reshape(EPAD // CHUNK, CHUNK)
    x_pad = jnp.pad(x, ((0, NPAD - N), (0, 0)))
    batch3 = jnp.concatenate([batch.astype(i32),
                              jnp.full((NPAD - N,), 1 << 20, i32)]).reshape(GRID, RB, 1)

    ones16 = jnp.asarray(_ONEHOT0)
    zeros16 = jnp.zeros((CHUNK, 16), _f32)
    zerosH = jnp.zeros((CHUNK, H), _f32)
    zerosZ = jnp.zeros((CHUNK, Z), _f32)

    degw = _sc_degree(dstm, ones16, zeros16)
    hw1p, dinv16 = _tc_encode(x_pad, W1, degw)
    p1 = _sc_aggregate(srcm, dstm, hw1p, zerosH, H)
    hw2p = _tc_mid(p1, hw1p, dinv16, b1.reshape(1, H), W2)
    p2 = _sc_aggregate(srcm, dstm, hw2p, zerosZ, Z)
    sums, cnts = _tc_pool(p2, hw2p, dinv16, b2.reshape(1, Z), batch3)
    out = _tc_head(sums, cnts, Wp, bp.reshape(1, NQ), qw,
                   jnp.asarray(_PW), jnp.asarray(_SGN), jnp.asarray(_CN),
                   jnp.asarray(_MEAS), Wc1[:Z], Wc1[Z:], bc1.reshape(1, 16),
                   Wc2, bc2.reshape(1, 1))
    return out
